# baseline (speedup 1.0000x reference)
import functools

import jax
import jax.numpy as jnp
from jax.experimental import pallas as pl
from jax.experimental.pallas import tpu as pltpu

_NUM_VECTOR = 8
_LAYER_IDX = 16
_ROWS = 1024
_TAIL = 8


def _copy_route_kernel(x_ref, xt_ref, w_ref, b_ref, vp_ref, out_ref, new_ref,
                       *, per_batch):
    j = pl.program_id(0)

    @pl.when(j == 0)
    def _route():
        token_act = xt_ref[:, _TAIL - 1, :]
        scores = jnp.dot(token_act, w_ref[...],
                         preferred_element_type=jnp.float32) + b_ref[...]
        probs = jax.nn.softmax(scores, axis=-1)
        idx = jnp.argmax(probs, axis=-1)
        keep = (idx == _NUM_VECTOR)[:, None]
        nb = token_act.shape[0]
        onehot = (jax.lax.broadcasted_iota(jnp.int32, (nb, _NUM_VECTOR), 1)
                  == jnp.minimum(idx, _NUM_VECTOR - 1)[:, None]).astype(jnp.float32)
        repl = jnp.dot(onehot, vp_ref[...],
                       preferred_element_type=jnp.float32)
        new_ref[...] = jnp.where(keep, token_act, repl)

    out_ref[...] = x_ref[...]

    @pl.when(j % per_batch == per_batch - 1)
    def _patch():
        b = j // per_batch
        out_ref[pl.ds(_ROWS - 1, 1), :] = new_ref[pl.ds(b, 1), :]


def kernel(x, vector_pool, gate_W, gate_b):
    B, S, H = x.shape
    vp16 = vector_pool[:, _LAYER_IDX, :]
    gate_b2 = gate_b.reshape(1, -1)
    x2 = x.reshape(B * S, H)
    nblk = (B * S) // _ROWS
    per_batch = S // _ROWS
    out2 = pl.pallas_call(
        functools.partial(_copy_route_kernel, per_batch=per_batch),
        grid=(nblk,),
        in_specs=[
            pl.BlockSpec((_ROWS, H), lambda j: (j, 0)),
            pl.BlockSpec((B, _TAIL, H), lambda j: (0, S // _TAIL - 1, 0)),
            pl.BlockSpec((H, _NUM_VECTOR + 1), lambda j: (0, 0)),
            pl.BlockSpec((1, _NUM_VECTOR + 1), lambda j: (0, 0)),
            pl.BlockSpec((_NUM_VECTOR, H), lambda j: (0, 0)),
        ],
        out_specs=pl.BlockSpec((_ROWS, H), lambda j: (j, 0)),
        out_shape=jax.ShapeDtypeStruct((B * S, H), x.dtype),
        scratch_shapes=[pltpu.VMEM((B, H), jnp.float32)],
    )(x2, x, gate_W, gate_b2, vp16)
    return out2.reshape(B, S, H)


# speedup vs baseline: 1.0053x; 1.0053x over previous
import functools

import jax
import jax.numpy as jnp
from jax.experimental import pallas as pl
from jax.experimental.pallas import tpu as pltpu

_NUM_VECTOR = 8
_LAYER_IDX = 16
_ROWS = 1024
_TAIL = 8


def _copy_route_kernel(x_ref, xt_ref, w_ref, b_ref, vp_ref, out_ref, new_ref,
                       *, per_batch):
    j = pl.program_id(0)

    @pl.when(j == 1)
    def _route():
        token_act = xt_ref[:, _TAIL - 1, :]
        scores = jnp.dot(token_act, w_ref[...],
                         preferred_element_type=jnp.float32) + b_ref[...]
        probs = jax.nn.softmax(scores, axis=-1)
        idx = jnp.argmax(probs, axis=-1)
        keep = (idx == _NUM_VECTOR)[:, None]
        nb = token_act.shape[0]
        onehot = (jax.lax.broadcasted_iota(jnp.int32, (nb, _NUM_VECTOR), 1)
                  == jnp.minimum(idx, _NUM_VECTOR - 1)[:, None]).astype(jnp.float32)
        repl = jnp.dot(onehot, vp_ref[...],
                       preferred_element_type=jnp.float32)
        new_ref[...] = jnp.where(keep, token_act, repl)

    out_ref[...] = x_ref[...]

    @pl.when(j % per_batch == per_batch - 1)
    def _patch():
        b = j // per_batch
        out_ref[pl.ds(_ROWS - 1, 1), :] = new_ref[pl.ds(b, 1), :]


def kernel(x, vector_pool, gate_W, gate_b):
    B, S, H = x.shape
    vp16 = vector_pool[:, _LAYER_IDX, :]
    gate_b2 = gate_b.reshape(1, -1)
    x2 = x.reshape(B * S, H)
    nblk = (B * S) // _ROWS
    per_batch = S // _ROWS
    out2 = pl.pallas_call(
        functools.partial(_copy_route_kernel, per_batch=per_batch),
        grid=(nblk,),
        in_specs=[
            pl.BlockSpec((_ROWS, H), lambda j: (j, 0)),
            pl.BlockSpec((B, _TAIL, H), lambda j: (0, S // _TAIL - 1, 0)),
            pl.BlockSpec((H, _NUM_VECTOR + 1), lambda j: (0, 0)),
            pl.BlockSpec((1, _NUM_VECTOR + 1), lambda j: (0, 0)),
            pl.BlockSpec((_NUM_VECTOR, H), lambda j: (0, 0)),
        ],
        out_specs=pl.BlockSpec((_ROWS, H), lambda j: (j, 0)),
        out_shape=jax.ShapeDtypeStruct((B * S, H), x.dtype),
        scratch_shapes=[pltpu.VMEM((B, H), jnp.float32)],
    )(x2, x, gate_W, gate_b2, vp16)
    return out2.reshape(B, S, H)
